# baseline (speedup 1.0000x reference)
import functools

import jax
import jax.numpy as jnp
from jax.experimental import pallas as pl
from jax.experimental.pallas import tpu as pltpu
from jax.experimental.pallas import tpu_sc as plsc

THR = 1e-20


def _pick_tile(E):
    for t in (200, 8):
        if E % t == 0:
            return t
    return E


def _attn_body(qq_ref, emb_ref, WihT_ref, WhhT_ref, bih_ref, bhh_ref,
               WlinT_ref, blin_ref, out_ref):
    B = qq_ref.shape[1]
    NQ1, EM = emb_ref.shape
    K, _, _, H4 = WihT_ref.shape
    H = H4 // 4

    qv = qq_ref[0, :]
    cols = jax.lax.broadcasted_iota(jnp.int32, (B, NQ1), 1)
    oh = jnp.where(cols == qv[:, None], 1.0, 0.0).astype(jnp.float32)
    q01 = jnp.dot(oh, emb_ref[...], preferred_element_type=jnp.float32)
    q2 = jnp.broadcast_to(emb_ref[NQ1 - 1:NQ1, :], (B, EM))
    xs_f = (q01, q01, q2)
    xs_b = (q2, q01, q01)

    for k in range(K):
        hs = [[None] * 3, [None] * 3]
        for d, xs in ((0, xs_f), (1, xs_b)):
            h = jnp.zeros((B, H), jnp.float32)
            c = jnp.zeros((B, H), jnp.float32)
            for t in range(3):
                g = (jnp.dot(xs[t], WihT_ref[k, d], preferred_element_type=jnp.float32)
                     + jnp.dot(h, WhhT_ref[k, d], preferred_element_type=jnp.float32)
                     + bih_ref[k, d] + bhh_ref[k, d])
                ig = jax.nn.sigmoid(g[:, 0:H])
                fg = jax.nn.sigmoid(g[:, H:2 * H])
                gg = jnp.tanh(g[:, 2 * H:3 * H])
                og = jax.nn.sigmoid(g[:, 3 * H:4 * H])
                c = fg * c + ig * gg
                h = og * jnp.tanh(c)
                hs[d][t] = h
        for t in range(2):
            out_t = jnp.concatenate([hs[0][t], hs[1][2 - t]], axis=1)
            logits = jnp.dot(out_t, WlinT_ref[...],
                             preferred_element_type=jnp.float32) + blin_ref[0, :]
            m = jnp.max(logits, axis=1, keepdims=True)
            e = jnp.exp(logits - m)
            a = e / jnp.sum(e, axis=1, keepdims=True)
            out_ref[k, t, :, :] = a


def _attention(qq, emb, Wih, Whh, bih, bhh, Wlin, blin):
    K, _, H4, EM = Wih.shape
    B = qq.shape[0]
    NOP1 = Wlin.shape[0]
    return pl.pallas_call(
        _attn_body,
        out_shape=jax.ShapeDtypeStruct((K, 2, B, NOP1), jnp.float32),
    )(qq.reshape(1, B).astype(jnp.int32), emb,
      Wih.transpose(0, 1, 3, 2), Whh.transpose(0, 1, 3, 2),
      bih.reshape(K, 2, 1, H4), bhh.reshape(K, 2, 1, H4),
      Wlin.T, blin.reshape(1, NOP1))


def _onehot_body(tt_ref, out_ref):
    T, C = out_ref.shape
    B = tt_ref.shape[1]
    i = pl.program_id(0)
    rows = i * T + jax.lax.broadcasted_iota(jnp.int32, (T, B), 0)
    m = jnp.where(rows == tt_ref[0, :][None, :], 1.0, 0.0).astype(jnp.float32)
    out_ref[...] = jnp.concatenate([m] * (C // B), axis=1)


def _onehot_rhs(tt, E, K):
    B = tt.shape[0]
    T = _pick_tile(E)
    return pl.pallas_call(
        _onehot_body,
        grid=(E // T,),
        in_specs=[pl.BlockSpec((1, B), lambda i: (0, 0))],
        out_specs=pl.BlockSpec((T, K * B), lambda i: (i, 0)),
        out_shape=jax.ShapeDtypeStruct((E, K * B), jnp.float32),
    )(tt.reshape(1, B).astype(jnp.int32))


def _sc_gather(tt, mdb):
    R, E, _ = mdb.shape
    B = tt.shape[0]
    E_pad = ((E + 2047) // 2048) * 2048
    mdb_flat = mdb.reshape(R * E * E)
    tt16 = jnp.broadcast_to(tt.astype(jnp.int32)[:, None], (B, 16))
    info = plsc.get_sparse_core_info()
    NW = info.num_cores * info.num_subcores
    per_w = (R * B) // NW
    mesh = plsc.VectorSubcoreMesh(core_axis_name="c", subcore_axis_name="s")

    @functools.partial(
        pl.kernel, mesh=mesh,
        out_type=jax.ShapeDtypeStruct((R, B, E_pad), jnp.float32),
        compiler_params=pltpu.CompilerParams(use_tc_tiling_on_sc=False),
        scratch_types=[
            pltpu.VMEM((16,), jnp.int32),
            pltpu.VMEM((E,), jnp.int32),
            pltpu.VMEM((E,), jnp.float32),
            pltpu.SemaphoreType.DMA,
        ],
    )
    def gk(tt16_hbm, flat_hbm, out_hbm, tts_v, idx_v, col_v, sem):
        wid = jax.lax.axis_index("s") * info.num_cores + jax.lax.axis_index("c")
        lane = jax.lax.iota(jnp.int32, 16)
        for j in range(per_w):
            p = wid * per_w + j
            r = p // B
            b = p % B
            pltpu.sync_copy(tt16_hbm.at[b], tts_v)
            base = tts_v[...] + r * (E * E)

            def build(g, _):
                idx_v[pl.ds(g * 16, 16)] = base + (g * 16 + lane) * E
                return 0

            jax.lax.fori_loop(0, E // 16, build, 0)
            pltpu.async_copy(flat_hbm.at[idx_v], col_v, sem).wait()
            pltpu.sync_copy(col_v, out_hbm.at[r, b, pl.ds(0, E)])

    out = gk(tt16, mdb_flat)
    return out.reshape(R, B, E_pad)


def _combine_body(E, Gt_ref, tt_ref, A0_ref, acc_ref, norms_ref):
    R = Gt_ref.shape[0]
    B = Gt_ref.shape[1]
    TS = Gt_ref.shape[2]
    K = A0_ref.shape[0] // B
    i = pl.program_id(0)
    lanes = i * TS + jax.lax.broadcasted_iota(jnp.int32, (B, TS), 1)
    ttv = tt_ref[0, :]
    onehot = jnp.where(lanes == ttv[:, None], 1.0, 0.0).astype(jnp.float32)
    lane_ok = lanes < E

    @pl.when(i == 0)
    def _():
        norms_ref[...] = jnp.zeros_like(norms_ref)

    for k in range(K):
        acc_k = onehot * A0_ref[k * B:(k + 1) * B, 4][:, None]
        for r in range(R):
            acc_k = acc_k + Gt_ref[r] * A0_ref[k * B:(k + 1) * B, r][:, None]
        contrib = jnp.sum(jnp.where(lane_ok, jnp.abs(acc_k), 0.0),
                          axis=1)[None, :]
        norms_ref[0:1, k * B:(k + 1) * B] += contrib
        acc_ref[:, k * B:(k + 1) * B] = acc_k.T


def _combine(Gt, tt, A0, E):
    R, B, E_pad = Gt.shape
    C = A0.shape[0]
    K = C // B
    TS = 256
    n = (E + TS - 1) // TS
    return pl.pallas_call(
        functools.partial(_combine_body, E),
        grid=(n,),
        in_specs=[
            pl.BlockSpec((R, B, TS), lambda i: (0, 0, i)),
            pl.BlockSpec((1, B), lambda i: (0, 0)),
            pl.BlockSpec((C, 5), lambda i: (0, 0)),
        ],
        out_specs=[
            pl.BlockSpec((TS, C), lambda i: (i, 0)),
            pl.BlockSpec((1, C), lambda i: (0, 0)),
        ],
        out_shape=[
            jax.ShapeDtypeStruct((E, C), jnp.float32),
            jax.ShapeDtypeStruct((1, C), jnp.float32),
        ],
        compiler_params=pltpu.CompilerParams(
            dimension_semantics=("arbitrary",)),
    )(Gt, tt.reshape(1, B).astype(jnp.int32), A0)


def _prop_body(mdb_ref, rhs_ref, Ar_ref, Askip_ref, normin_ref,
               acc_ref, norms_ref):
    T = acc_ref.shape[0]
    i = pl.program_id(0)
    r = pl.program_id(1)
    R = pl.num_programs(1)
    ninv = 1.0 / jnp.maximum(normin_ref[0, :], THR)
    w = Ar_ref[0, 0, :] * ninv
    part = jnp.dot(mdb_ref[0], rhs_ref[...],
                   preferred_element_type=jnp.float32) * w[None, :]

    @pl.when(r == 0)
    def _():
        wskip = Askip_ref[0, :] * ninv
        acc_ref[...] = part + rhs_ref[pl.ds(i * T, T), :] * wskip[None, :]

    @pl.when(r > 0)
    def _():
        acc_ref[...] += part

    @pl.when(r == R - 1)
    def _():
        colsum = jnp.sum(jnp.abs(acc_ref[...]), axis=0, keepdims=True)

        @pl.when(i == 0)
        def _():
            norms_ref[...] = colsum

        @pl.when(i > 0)
        def _():
            norms_ref[...] += colsum


def _propagate(mdb, rhs, A, norm_in):
    R, E, _ = mdb.shape
    C = rhs.shape[1]
    T = _pick_tile(E)
    Ar = A.T.reshape(R + 1, 1, C)
    return pl.pallas_call(
        _prop_body,
        grid=(E // T, R),
        in_specs=[
            pl.BlockSpec((1, T, E), lambda i, r: (r, i, 0)),
            pl.BlockSpec((E, C), lambda i, r: (0, 0)),
            pl.BlockSpec((1, 1, C), lambda i, r: (r, 0, 0)),
            pl.BlockSpec((1, C), lambda i, r: (0, 0)),
            pl.BlockSpec((1, C), lambda i, r: (0, 0)),
        ],
        out_specs=[
            pl.BlockSpec((T, C), lambda i, r: (i, 0)),
            pl.BlockSpec((1, C), lambda i, r: (0, 0)),
        ],
        out_shape=[
            jax.ShapeDtypeStruct((E, C), jnp.float32),
            jax.ShapeDtypeStruct((1, C), jnp.float32),
        ],
        compiler_params=pltpu.CompilerParams(
            dimension_semantics=("arbitrary", "arbitrary")),
    )(mdb, rhs, Ar[:R], Ar[R], norm_in)


def _epi_body(acc_ref, norms_ref, out_ref):
    B = out_ref.shape[1]
    K = acc_ref.shape[1] // B
    t = acc_ref[...] / jnp.maximum(norms_ref[0, :], THR)[None, :]
    s = t[:, 0:B]
    for k in range(1, K):
        s = s + t[:, k * B:(k + 1) * B]
    out_ref[...] = s


def _epilogue(acc, norms, B):
    E, C = acc.shape
    T = _pick_tile(E)
    return pl.pallas_call(
        _epi_body,
        grid=(E // T,),
        in_specs=[
            pl.BlockSpec((T, C), lambda i: (i, 0)),
            pl.BlockSpec((1, C), lambda i: (0, 0)),
        ],
        out_specs=pl.BlockSpec((T, B), lambda i: (i, 0)),
        out_shape=jax.ShapeDtypeStruct((E, B), jnp.float32),
    )(acc, norms)


def kernel(qq, tt, mdb, emb, Wih, Whh, bih, bhh, Wlin, blin):
    R, E, _ = mdb.shape
    B = qq.shape[0]
    K = Wih.shape[0]
    C = K * B

    attn = _attention(qq, emb, Wih, Whh, bih, bhh, Wlin, blin)
    A0 = attn[:, 0].reshape(C, R + 1)
    A1 = attn[:, 1].reshape(C, R + 1)

    info = plsc.get_sparse_core_info()
    NW = info.num_cores * info.num_subcores
    sc_ok = (E % 16 == 0) and ((R * B) % NW == 0)
    if sc_ok:
        Gt = _sc_gather(tt, mdb)
        acc0, norms0 = _combine(Gt, tt, A0, E)
    else:
        rhs0 = _onehot_rhs(tt, E, K)
        ones = jnp.ones((1, C), jnp.float32)
        acc0, norms0 = _propagate(mdb, rhs0, A0, ones)
    acc1, norms1 = _propagate(mdb, acc0, A1, norms0)
    predT = _epilogue(acc1, norms1, B)
    return predT.T


# speedup vs baseline: 2.0261x; 2.0261x over previous
import functools

import jax
import jax.numpy as jnp
from jax.experimental import pallas as pl
from jax.experimental.pallas import tpu as pltpu
from jax.experimental.pallas import tpu_sc as plsc

THR = 1e-20


def _pick_tile(E):
    for t in (200, 8):
        if E % t == 0:
            return t
    return E


def _attn_body(qq_ref, emb_ref, WihT_ref, WhhT_ref, bih_ref, bhh_ref,
               WlinT_ref, blin_ref, out_ref):
    B = qq_ref.shape[1]
    NQ1, EM = emb_ref.shape
    K, _, _, H4 = WihT_ref.shape
    H = H4 // 4

    qv = qq_ref[0, :]
    cols = jax.lax.broadcasted_iota(jnp.int32, (B, NQ1), 1)
    oh = jnp.where(cols == qv[:, None], 1.0, 0.0).astype(jnp.float32)
    q01 = jnp.dot(oh, emb_ref[...], preferred_element_type=jnp.float32)
    q2 = jnp.broadcast_to(emb_ref[NQ1 - 1:NQ1, :], (B, EM))
    xs_f = (q01, q01, q2)
    xs_b = (q2, q01, q01)

    for k in range(K):
        hs = [[None] * 3, [None] * 3]
        for d, xs in ((0, xs_f), (1, xs_b)):
            h = jnp.zeros((B, H), jnp.float32)
            c = jnp.zeros((B, H), jnp.float32)
            for t in range(3):
                g = (jnp.dot(xs[t], WihT_ref[k, d], preferred_element_type=jnp.float32)
                     + jnp.dot(h, WhhT_ref[k, d], preferred_element_type=jnp.float32)
                     + bih_ref[k, d] + bhh_ref[k, d])
                ig = jax.nn.sigmoid(g[:, 0:H])
                fg = jax.nn.sigmoid(g[:, H:2 * H])
                gg = jnp.tanh(g[:, 2 * H:3 * H])
                og = jax.nn.sigmoid(g[:, 3 * H:4 * H])
                c = fg * c + ig * gg
                h = og * jnp.tanh(c)
                hs[d][t] = h
        for t in range(2):
            out_t = jnp.concatenate([hs[0][t], hs[1][2 - t]], axis=1)
            logits = jnp.dot(out_t, WlinT_ref[...],
                             preferred_element_type=jnp.float32) + blin_ref[0, :]
            m = jnp.max(logits, axis=1, keepdims=True)
            e = jnp.exp(logits - m)
            a = e / jnp.sum(e, axis=1, keepdims=True)
            out_ref[k, t, :, :] = a


def _attention(qq, emb, Wih, Whh, bih, bhh, Wlin, blin):
    K, _, H4, EM = Wih.shape
    B = qq.shape[0]
    NOP1 = Wlin.shape[0]
    return pl.pallas_call(
        _attn_body,
        out_shape=jax.ShapeDtypeStruct((K, 2, B, NOP1), jnp.float32),
    )(qq.reshape(1, B).astype(jnp.int32), emb,
      Wih.transpose(0, 1, 3, 2), Whh.transpose(0, 1, 3, 2),
      bih.reshape(K, 2, 1, H4), bhh.reshape(K, 2, 1, H4),
      Wlin.T, blin.reshape(1, NOP1))


def _onehot_body(tt_ref, out_ref):
    T, C = out_ref.shape
    B = tt_ref.shape[1]
    i = pl.program_id(0)
    rows = i * T + jax.lax.broadcasted_iota(jnp.int32, (T, B), 0)
    m = jnp.where(rows == tt_ref[0, :][None, :], 1.0, 0.0).astype(jnp.float32)
    out_ref[...] = jnp.concatenate([m] * (C // B), axis=1)


def _onehot_rhs(tt, E, K):
    B = tt.shape[0]
    T = _pick_tile(E)
    return pl.pallas_call(
        _onehot_body,
        grid=(E // T,),
        in_specs=[pl.BlockSpec((1, B), lambda i: (0, 0))],
        out_specs=pl.BlockSpec((T, K * B), lambda i: (i, 0)),
        out_shape=jax.ShapeDtypeStruct((E, K * B), jnp.float32),
    )(tt.reshape(1, B).astype(jnp.int32))


def _sc_gather(tt, mdb):
    R, E, _ = mdb.shape
    B = tt.shape[0]
    E_pad = ((E + 2047) // 2048) * 2048
    mdb_flat = mdb.reshape(R * E * E)
    tt16 = jnp.broadcast_to(tt.astype(jnp.int32)[:, None], (B, 16))
    info = plsc.get_sparse_core_info()
    NW = info.num_cores * info.num_subcores
    per_w = (R * B) // NW
    mesh = plsc.VectorSubcoreMesh(core_axis_name="c", subcore_axis_name="s")

    @functools.partial(
        pl.kernel, mesh=mesh,
        out_type=jax.ShapeDtypeStruct((R, B, E_pad), jnp.float32),
        compiler_params=pltpu.CompilerParams(use_tc_tiling_on_sc=False),
        scratch_types=[
            pltpu.VMEM((16,), jnp.int32),
            pltpu.VMEM((E,), jnp.int32),
            pltpu.VMEM((E,), jnp.float32),
            pltpu.SemaphoreType.DMA,
        ],
    )
    def gk(tt16_hbm, flat_hbm, out_hbm, tts_v, idx_v, col_v, sem):
        wid = jax.lax.axis_index("s") * info.num_cores + jax.lax.axis_index("c")
        lane = jax.lax.iota(jnp.int32, 16)
        for j in range(per_w):
            p = wid * per_w + j
            r = p // B
            b = p % B
            pltpu.sync_copy(tt16_hbm.at[b], tts_v)
            base = tts_v[...] + r * (E * E)

            def build(g, _):
                idx_v[pl.ds(g * 16, 16)] = base + (g * 16 + lane) * E
                return 0

            jax.lax.fori_loop(0, E // 16, build, 0)
            pltpu.async_copy(flat_hbm.at[idx_v], col_v, sem).wait()
            pltpu.sync_copy(col_v, out_hbm.at[r, b, pl.ds(0, E)])

    out = gk(tt16, mdb_flat)
    return out.reshape(R, B, E_pad)


def _tcg_body(tt_ref, mdb_ref, out_ref, c0, c1, sin0, sin1, sout0, sout1):
    R = out_ref.shape[0]
    B = out_ref.shape[2]
    N = R * B
    r = pl.program_id(0)
    b = pl.program_id(1)
    s = r * B + b
    col = tt_ref[b]
    bn = (b + 1) % B
    rn = r + (b + 1) // B
    coln = tt_ref[bn]
    par = s % 2

    for p in range(2):
        @pl.when(par == p)
        def _(p=p):
            buf, sin, sout = (c0, sin0, sout0) if p == 0 else (c1, sin1, sout1)
            obuf, osin, osout = (c1, sin1, sout1) if p == 0 else (c0, sin0, sout0)

            @pl.when(s == 0)
            def _():
                pltpu.make_async_copy(
                    mdb_ref.at[r, :, pl.ds(col, 1)], buf, sin).start()

            @pl.when(jnp.logical_and(s >= 1, s + 1 < N))
            def _():
                pltpu.make_async_copy(
                    obuf, out_ref.at[r, :, pl.ds(b, 1)], osout).wait()

            @pl.when(s + 1 < N)
            def _():
                pltpu.make_async_copy(
                    mdb_ref.at[rn, :, pl.ds(coln, 1)], obuf, osin).start()

            pltpu.make_async_copy(
                mdb_ref.at[r, :, pl.ds(col, 1)], buf, sin).wait()
            pltpu.make_async_copy(
                buf, out_ref.at[r, :, pl.ds(b, 1)], sout).start()

            @pl.when(s == N - 1)
            def _():
                pltpu.make_async_copy(
                    buf, out_ref.at[r, :, pl.ds(b, 1)], sout).wait()
                pltpu.make_async_copy(
                    obuf, out_ref.at[r, :, pl.ds(b, 1)], osout).wait()


def _tc_gather(tt, mdb):
    R, E, _ = mdb.shape
    B = tt.shape[0]
    return pl.pallas_call(
        _tcg_body,
        grid=(R, B),
        in_specs=[
            pl.BlockSpec(memory_space=pltpu.SMEM),
            pl.BlockSpec(memory_space=pl.ANY),
        ],
        out_specs=pl.BlockSpec(memory_space=pl.ANY),
        out_shape=jax.ShapeDtypeStruct((R, E, B), jnp.float32),
        scratch_shapes=[
            pltpu.VMEM((E, 1), jnp.float32),
            pltpu.VMEM((E, 1), jnp.float32),
            pltpu.SemaphoreType.DMA,
            pltpu.SemaphoreType.DMA,
            pltpu.SemaphoreType.DMA,
            pltpu.SemaphoreType.DMA,
        ],
        compiler_params=pltpu.CompilerParams(
            dimension_semantics=("arbitrary", "arbitrary")),
    )(tt.astype(jnp.int32), mdb)


def _combine_body(G_ref, tt_ref, A0T_ref, acc_ref, norms_ref):
    R = G_ref.shape[0]
    T = G_ref.shape[1]
    B = G_ref.shape[2]
    K = A0T_ref.shape[1] // B
    i = pl.program_id(0)
    rows = i * T + jax.lax.broadcasted_iota(jnp.int32, (T, B), 0)
    ttv = tt_ref[0, :]
    onehot = jnp.where(rows == ttv[None, :], 1.0, 0.0).astype(jnp.float32)

    @pl.when(i == 0)
    def _():
        norms_ref[...] = jnp.zeros_like(norms_ref)

    for k in range(K):
        sl = pl.ds(k * B, B)
        acc_k = onehot * A0T_ref[4, sl][None, :]
        for r in range(R):
            acc_k = acc_k + G_ref[r] * A0T_ref[r, sl][None, :]
        acc_ref[:, sl] = acc_k
        norms_ref[0:1, sl] += jnp.sum(jnp.abs(acc_k), axis=0)[None, :]


def _combine(G, tt, A0T):
    R, E, B = G.shape
    C = A0T.shape[1]
    T = _pick_tile(E)
    return pl.pallas_call(
        _combine_body,
        grid=(E // T,),
        in_specs=[
            pl.BlockSpec((R, T, B), lambda i: (0, i, 0)),
            pl.BlockSpec((1, B), lambda i: (0, 0)),
            pl.BlockSpec((5, C), lambda i: (0, 0)),
        ],
        out_specs=[
            pl.BlockSpec((T, C), lambda i: (i, 0)),
            pl.BlockSpec((1, C), lambda i: (0, 0)),
        ],
        out_shape=[
            jax.ShapeDtypeStruct((E, C), jnp.float32),
            jax.ShapeDtypeStruct((1, C), jnp.float32),
        ],
        compiler_params=pltpu.CompilerParams(
            dimension_semantics=("arbitrary",)),
    )(G, tt.reshape(1, B).astype(jnp.int32), A0T)


def _prop_body(mdb_ref, rhs_ref, Ar_ref, Askip_ref, normin_ref,
               acc_ref, norms_ref):
    T = acc_ref.shape[0]
    i = pl.program_id(0)
    r = pl.program_id(1)
    R = pl.num_programs(1)
    ninv = 1.0 / jnp.maximum(normin_ref[0, :], THR)
    w = Ar_ref[0, 0, :] * ninv
    part = jnp.dot(mdb_ref[0].astype(jnp.bfloat16),
                   rhs_ref[...].astype(jnp.bfloat16),
                   preferred_element_type=jnp.float32) * w[None, :]

    @pl.when(r == 0)
    def _():
        wskip = Askip_ref[0, :] * ninv
        acc_ref[...] = part + rhs_ref[pl.ds(i * T, T), :] * wskip[None, :]

    @pl.when(r > 0)
    def _():
        acc_ref[...] += part

    @pl.when(r == R - 1)
    def _():
        colsum = jnp.sum(jnp.abs(acc_ref[...]), axis=0, keepdims=True)

        @pl.when(i == 0)
        def _():
            norms_ref[...] = colsum

        @pl.when(i > 0)
        def _():
            norms_ref[...] += colsum


def _propagate(mdb, rhs, A, norm_in):
    R, E, _ = mdb.shape
    C = rhs.shape[1]
    T = _pick_tile(E)
    Ar = A.T.reshape(R + 1, 1, C)
    return pl.pallas_call(
        _prop_body,
        grid=(E // T, R),
        in_specs=[
            pl.BlockSpec((1, T, E), lambda i, r: (r, i, 0)),
            pl.BlockSpec((E, C), lambda i, r: (0, 0)),
            pl.BlockSpec((1, 1, C), lambda i, r: (r, 0, 0)),
            pl.BlockSpec((1, C), lambda i, r: (0, 0)),
            pl.BlockSpec((1, C), lambda i, r: (0, 0)),
        ],
        out_specs=[
            pl.BlockSpec((T, C), lambda i, r: (i, 0)),
            pl.BlockSpec((1, C), lambda i, r: (0, 0)),
        ],
        out_shape=[
            jax.ShapeDtypeStruct((E, C), jnp.float32),
            jax.ShapeDtypeStruct((1, C), jnp.float32),
        ],
        compiler_params=pltpu.CompilerParams(
            dimension_semantics=("arbitrary", "arbitrary")),
    )(mdb, rhs, Ar[:R], Ar[R], norm_in)


def _epi_body(acc_ref, norms_ref, out_ref):
    B = out_ref.shape[1]
    K = acc_ref.shape[1] // B
    t = acc_ref[...] / jnp.maximum(norms_ref[0, :], THR)[None, :]
    s = t[:, 0:B]
    for k in range(1, K):
        s = s + t[:, k * B:(k + 1) * B]
    out_ref[...] = s


def _epilogue(acc, norms, B):
    E, C = acc.shape
    T = _pick_tile(E)
    return pl.pallas_call(
        _epi_body,
        grid=(E // T,),
        in_specs=[
            pl.BlockSpec((T, C), lambda i: (i, 0)),
            pl.BlockSpec((1, C), lambda i: (0, 0)),
        ],
        out_specs=pl.BlockSpec((T, B), lambda i: (i, 0)),
        out_shape=jax.ShapeDtypeStruct((E, B), jnp.float32),
    )(acc, norms)


def kernel(qq, tt, mdb, emb, Wih, Whh, bih, bhh, Wlin, blin):
    R, E, _ = mdb.shape
    B = qq.shape[0]
    K = Wih.shape[0]
    C = K * B

    attn = _attention(qq, emb, Wih, Whh, bih, bhh, Wlin, blin)
    A0 = attn[:, 0].reshape(C, R + 1)
    A1 = attn[:, 1].reshape(C, R + 1)

    rhs0 = _onehot_rhs(tt, E, K)
    ones = jnp.ones((1, C), jnp.float32)
    acc0, norms0 = _propagate(mdb, rhs0, A0, ones)
    acc1, norms1 = _propagate(mdb, acc0, A1, norms0)
    predT = _epilogue(acc1, norms1, B)
    return predT.T


# speedup vs baseline: 2.1436x; 1.0580x over previous
import functools

import jax
import jax.numpy as jnp
from jax.experimental import pallas as pl
from jax.experimental.pallas import tpu as pltpu
from jax.experimental.pallas import tpu_sc as plsc

THR = 1e-20


def _pick_tile(E):
    for t in (400, 200, 8):
        if E % t == 0:
            return t
    return E


def _attn_body(qq_ref, emb_ref, WihT_ref, WhhT_ref, bih_ref, bhh_ref,
               WlinT_ref, blin_ref, out_ref):
    B = qq_ref.shape[1]
    NQ1, EM = emb_ref.shape
    K, _, _, H4 = WihT_ref.shape
    H = H4 // 4

    qv = qq_ref[0, :]
    cols = jax.lax.broadcasted_iota(jnp.int32, (B, NQ1), 1)
    oh = jnp.where(cols == qv[:, None], 1.0, 0.0).astype(jnp.float32)
    q01 = jnp.dot(oh, emb_ref[...], preferred_element_type=jnp.float32)
    q2 = jnp.broadcast_to(emb_ref[NQ1 - 1:NQ1, :], (B, EM))
    xs_f = (q01, q01, q2)
    xs_b = (q2, q01, q01)

    for k in range(K):
        hs = [[None] * 3, [None] * 3]
        for d, xs in ((0, xs_f), (1, xs_b)):
            h = jnp.zeros((B, H), jnp.float32)
            c = jnp.zeros((B, H), jnp.float32)
            for t in range(3):
                g = (jnp.dot(xs[t], WihT_ref[k, d], preferred_element_type=jnp.float32)
                     + jnp.dot(h, WhhT_ref[k, d], preferred_element_type=jnp.float32)
                     + bih_ref[k, d] + bhh_ref[k, d])
                ig = jax.nn.sigmoid(g[:, 0:H])
                fg = jax.nn.sigmoid(g[:, H:2 * H])
                gg = jnp.tanh(g[:, 2 * H:3 * H])
                og = jax.nn.sigmoid(g[:, 3 * H:4 * H])
                c = fg * c + ig * gg
                h = og * jnp.tanh(c)
                hs[d][t] = h
        for t in range(2):
            out_t = jnp.concatenate([hs[0][t], hs[1][2 - t]], axis=1)
            logits = jnp.dot(out_t, WlinT_ref[...],
                             preferred_element_type=jnp.float32) + blin_ref[0, :]
            m = jnp.max(logits, axis=1, keepdims=True)
            e = jnp.exp(logits - m)
            a = e / jnp.sum(e, axis=1, keepdims=True)
            out_ref[k, t, :, :] = a


def _attention(qq, emb, Wih, Whh, bih, bhh, Wlin, blin):
    K, _, H4, EM = Wih.shape
    B = qq.shape[0]
    NOP1 = Wlin.shape[0]
    return pl.pallas_call(
        _attn_body,
        out_shape=jax.ShapeDtypeStruct((K, 2, B, NOP1), jnp.float32),
    )(qq.reshape(1, B).astype(jnp.int32), emb,
      Wih.transpose(0, 1, 3, 2), Whh.transpose(0, 1, 3, 2),
      bih.reshape(K, 2, 1, H4), bhh.reshape(K, 2, 1, H4),
      Wlin.T, blin.reshape(1, NOP1))


def _onehot_body(tt_ref, out_ref):
    T, C = out_ref.shape
    B = tt_ref.shape[1]
    i = pl.program_id(0)
    rows = i * T + jax.lax.broadcasted_iota(jnp.int32, (T, B), 0)
    m = jnp.where(rows == tt_ref[0, :][None, :], 1.0, 0.0).astype(jnp.float32)
    out_ref[...] = jnp.concatenate([m] * (C // B), axis=1)


def _onehot_rhs(tt, E, K):
    B = tt.shape[0]
    T = _pick_tile(E)
    return pl.pallas_call(
        _onehot_body,
        grid=(E // T,),
        in_specs=[pl.BlockSpec((1, B), lambda i: (0, 0))],
        out_specs=pl.BlockSpec((T, K * B), lambda i: (i, 0)),
        out_shape=jax.ShapeDtypeStruct((E, K * B), jnp.float32),
    )(tt.reshape(1, B).astype(jnp.int32))


def _sc_gather(tt, mdb):
    R, E, _ = mdb.shape
    B = tt.shape[0]
    E_pad = ((E + 2047) // 2048) * 2048
    mdb_flat = mdb.reshape(R * E * E)
    tt16 = jnp.broadcast_to(tt.astype(jnp.int32)[:, None], (B, 16))
    info = plsc.get_sparse_core_info()
    NW = info.num_cores * info.num_subcores
    per_w = (R * B) // NW
    mesh = plsc.VectorSubcoreMesh(core_axis_name="c", subcore_axis_name="s")

    @functools.partial(
        pl.kernel, mesh=mesh,
        out_type=jax.ShapeDtypeStruct((R, B, E_pad), jnp.float32),
        compiler_params=pltpu.CompilerParams(use_tc_tiling_on_sc=False),
        scratch_types=[
            pltpu.VMEM((16,), jnp.int32),
            pltpu.VMEM((E,), jnp.int32),
            pltpu.VMEM((E,), jnp.float32),
            pltpu.SemaphoreType.DMA,
        ],
    )
    def gk(tt16_hbm, flat_hbm, out_hbm, tts_v, idx_v, col_v, sem):
        wid = jax.lax.axis_index("s") * info.num_cores + jax.lax.axis_index("c")
        lane = jax.lax.iota(jnp.int32, 16)
        for j in range(per_w):
            p = wid * per_w + j
            r = p // B
            b = p % B
            pltpu.sync_copy(tt16_hbm.at[b], tts_v)
            base = tts_v[...] + r * (E * E)

            def build(g, _):
                idx_v[pl.ds(g * 16, 16)] = base + (g * 16 + lane) * E
                return 0

            jax.lax.fori_loop(0, E // 16, build, 0)
            pltpu.async_copy(flat_hbm.at[idx_v], col_v, sem).wait()
            pltpu.sync_copy(col_v, out_hbm.at[r, b, pl.ds(0, E)])

    out = gk(tt16, mdb_flat)
    return out.reshape(R, B, E_pad)


def _tcg_body(tt_ref, mdb_ref, out_ref, c0, c1, sin0, sin1, sout0, sout1):
    R = out_ref.shape[0]
    B = out_ref.shape[2]
    N = R * B
    r = pl.program_id(0)
    b = pl.program_id(1)
    s = r * B + b
    col = tt_ref[b]
    bn = (b + 1) % B
    rn = r + (b + 1) // B
    coln = tt_ref[bn]
    par = s % 2

    for p in range(2):
        @pl.when(par == p)
        def _(p=p):
            buf, sin, sout = (c0, sin0, sout0) if p == 0 else (c1, sin1, sout1)
            obuf, osin, osout = (c1, sin1, sout1) if p == 0 else (c0, sin0, sout0)

            @pl.when(s == 0)
            def _():
                pltpu.make_async_copy(
                    mdb_ref.at[r, :, pl.ds(col, 1)], buf, sin).start()

            @pl.when(jnp.logical_and(s >= 1, s + 1 < N))
            def _():
                pltpu.make_async_copy(
                    obuf, out_ref.at[r, :, pl.ds(b, 1)], osout).wait()

            @pl.when(s + 1 < N)
            def _():
                pltpu.make_async_copy(
                    mdb_ref.at[rn, :, pl.ds(coln, 1)], obuf, osin).start()

            pltpu.make_async_copy(
                mdb_ref.at[r, :, pl.ds(col, 1)], buf, sin).wait()
            pltpu.make_async_copy(
                buf, out_ref.at[r, :, pl.ds(b, 1)], sout).start()

            @pl.when(s == N - 1)
            def _():
                pltpu.make_async_copy(
                    buf, out_ref.at[r, :, pl.ds(b, 1)], sout).wait()
                pltpu.make_async_copy(
                    obuf, out_ref.at[r, :, pl.ds(b, 1)], osout).wait()


def _tc_gather(tt, mdb):
    R, E, _ = mdb.shape
    B = tt.shape[0]
    return pl.pallas_call(
        _tcg_body,
        grid=(R, B),
        in_specs=[
            pl.BlockSpec(memory_space=pltpu.SMEM),
            pl.BlockSpec(memory_space=pl.ANY),
        ],
        out_specs=pl.BlockSpec(memory_space=pl.ANY),
        out_shape=jax.ShapeDtypeStruct((R, E, B), jnp.float32),
        scratch_shapes=[
            pltpu.VMEM((E, 1), jnp.float32),
            pltpu.VMEM((E, 1), jnp.float32),
            pltpu.SemaphoreType.DMA,
            pltpu.SemaphoreType.DMA,
            pltpu.SemaphoreType.DMA,
            pltpu.SemaphoreType.DMA,
        ],
        compiler_params=pltpu.CompilerParams(
            dimension_semantics=("arbitrary", "arbitrary")),
    )(tt.astype(jnp.int32), mdb)


def _combine_body(G_ref, tt_ref, A0T_ref, acc_ref, norms_ref):
    R = G_ref.shape[0]
    T = G_ref.shape[1]
    B = G_ref.shape[2]
    K = A0T_ref.shape[1] // B
    i = pl.program_id(0)
    rows = i * T + jax.lax.broadcasted_iota(jnp.int32, (T, B), 0)
    ttv = tt_ref[0, :]
    onehot = jnp.where(rows == ttv[None, :], 1.0, 0.0).astype(jnp.float32)

    @pl.when(i == 0)
    def _():
        norms_ref[...] = jnp.zeros_like(norms_ref)

    for k in range(K):
        sl = pl.ds(k * B, B)
        acc_k = onehot * A0T_ref[4, sl][None, :]
        for r in range(R):
            acc_k = acc_k + G_ref[r] * A0T_ref[r, sl][None, :]
        acc_ref[:, sl] = acc_k
        norms_ref[0:1, sl] += jnp.sum(jnp.abs(acc_k), axis=0)[None, :]


def _combine(G, tt, A0T):
    R, E, B = G.shape
    C = A0T.shape[1]
    T = _pick_tile(E)
    return pl.pallas_call(
        _combine_body,
        grid=(E // T,),
        in_specs=[
            pl.BlockSpec((R, T, B), lambda i: (0, i, 0)),
            pl.BlockSpec((1, B), lambda i: (0, 0)),
            pl.BlockSpec((5, C), lambda i: (0, 0)),
        ],
        out_specs=[
            pl.BlockSpec((T, C), lambda i: (i, 0)),
            pl.BlockSpec((1, C), lambda i: (0, 0)),
        ],
        out_shape=[
            jax.ShapeDtypeStruct((E, C), jnp.float32),
            jax.ShapeDtypeStruct((1, C), jnp.float32),
        ],
        compiler_params=pltpu.CompilerParams(
            dimension_semantics=("arbitrary",)),
    )(G, tt.reshape(1, B).astype(jnp.int32), A0T)


def _prop_body(mdb_ref, rhs_ref, Ar_ref, Askip_ref, normin_ref,
               acc_ref, norms_ref):
    T = acc_ref.shape[0]
    i = pl.program_id(0)
    r = pl.program_id(1)
    R = pl.num_programs(1)
    ninv = 1.0 / jnp.maximum(normin_ref[0, :], THR)
    w = Ar_ref[0, 0, :] * ninv
    part = jnp.dot(mdb_ref[0], rhs_ref[...],
                   preferred_element_type=jnp.float32) * w[None, :]

    @pl.when(r == 0)
    def _():
        wskip = Askip_ref[0, :] * ninv
        acc_ref[...] = part + rhs_ref[pl.ds(i * T, T), :] * wskip[None, :]

    @pl.when(r > 0)
    def _():
        acc_ref[...] += part

    @pl.when(r == R - 1)
    def _():
        colsum = jnp.sum(jnp.abs(acc_ref[...]), axis=0, keepdims=True)

        @pl.when(i == 0)
        def _():
            norms_ref[...] = colsum

        @pl.when(i > 0)
        def _():
            norms_ref[...] += colsum


def _propagate(mdb, rhs, A, norm_in):
    R, E, _ = mdb.shape
    C = rhs.shape[1]
    T = _pick_tile(E)
    Ar = A.T.reshape(R + 1, 1, C)
    return pl.pallas_call(
        _prop_body,
        grid=(E // T, R),
        in_specs=[
            pl.BlockSpec((1, T, E), lambda i, r: (r, i, 0)),
            pl.BlockSpec((E, C), lambda i, r: (0, 0)),
            pl.BlockSpec((1, 1, C), lambda i, r: (r, 0, 0)),
            pl.BlockSpec((1, C), lambda i, r: (0, 0)),
            pl.BlockSpec((1, C), lambda i, r: (0, 0)),
        ],
        out_specs=[
            pl.BlockSpec((T, C), lambda i, r: (i, 0)),
            pl.BlockSpec((1, C), lambda i, r: (0, 0)),
        ],
        out_shape=[
            jax.ShapeDtypeStruct((E, C), jnp.float32),
            jax.ShapeDtypeStruct((1, C), jnp.float32),
        ],
        compiler_params=pltpu.CompilerParams(
            dimension_semantics=("arbitrary", "arbitrary")),
    )(mdb, rhs, Ar[:R], Ar[R], norm_in)


def _epi_body(acc_ref, norms_ref, out_ref):
    B = out_ref.shape[1]
    K = acc_ref.shape[1] // B
    t = acc_ref[...] / jnp.maximum(norms_ref[0, :], THR)[None, :]
    s = t[:, 0:B]
    for k in range(1, K):
        s = s + t[:, k * B:(k + 1) * B]
    out_ref[...] = s


def _epilogue(acc, norms, B):
    E, C = acc.shape
    T = _pick_tile(E)
    return pl.pallas_call(
        _epi_body,
        grid=(E // T,),
        in_specs=[
            pl.BlockSpec((T, C), lambda i: (i, 0)),
            pl.BlockSpec((1, C), lambda i: (0, 0)),
        ],
        out_specs=pl.BlockSpec((T, B), lambda i: (i, 0)),
        out_shape=jax.ShapeDtypeStruct((E, B), jnp.float32),
    )(acc, norms)


def kernel(qq, tt, mdb, emb, Wih, Whh, bih, bhh, Wlin, blin):
    R, E, _ = mdb.shape
    B = qq.shape[0]
    K = Wih.shape[0]
    C = K * B

    attn = _attention(qq, emb, Wih, Whh, bih, bhh, Wlin, blin)
    A0 = attn[:, 0].reshape(C, R + 1)
    A1 = attn[:, 1].reshape(C, R + 1)

    rhs0 = _onehot_rhs(tt, E, K)
    ones = jnp.ones((1, C), jnp.float32)
    acc0, norms0 = _propagate(mdb, rhs0, A0, ones)
    acc1, norms1 = _propagate(mdb, acc0, A1, norms0)
    predT = _epilogue(acc1, norms1, B)
    return predT.T
